# baseline (speedup 1.0000x reference)
import functools

import jax
import jax.numpy as jnp
from jax import lax
from jax.experimental import pallas as pl
from jax.experimental.pallas import tpu as pltpu
from jax.experimental.pallas import tpu_sc as plsc

N = 10000
E = 160000
NC = 2
NS = 16
EPT = E // NS
K = 128
CH = 80
EPD = E // (NC * NS)
CHD = 40
NA = N + 240
RPT = 624
TAIL = N - NS * RPT
ZB = 104


def _fill_rows(ref, rows, cols, value, dtype):
    per_row = cols // 16

    def body(i, _):
        r = i // per_row
        c = (i % per_row) * 16
        ref[r, pl.ds(c, 16)] = jnp.full((16,), value, dtype)
        return 0

    lax.fori_loop(0, rows * per_row, body, 0)


def _sc_mesh():
    return plsc.VectorSubcoreMesh(core_axis_name="c", subcore_axis_name="s")


def _zero_acc_slice(zsrc, acc, sid):
    def zero_chunk(i, _):
        pltpu.sync_copy(zsrc.at[pl.ds(0, 96)],
                        acc.at[pl.ds(sid * RPT + i * 96, 96)])
        return 0

    lax.fori_loop(0, 6, zero_chunk, 0)
    pltpu.sync_copy(zsrc.at[pl.ds(0, 48)], acc.at[pl.ds(sid * RPT + 576, 48)])

    @pl.when(sid == NS - 1)
    def _():
        pltpu.sync_copy(zsrc.at[pl.ds(0, TAIL)], acc.at[pl.ds(NS * RPT, TAIL)])


def _deg_kernel(dst_d):
    @functools.partial(
        pl.kernel,
        out_type=jax.ShapeDtypeStruct((2 * N, 128), jnp.float32),
        mesh=_sc_mesh(),
        scratch_types=[
            pltpu.VMEM((CHD, K), jnp.int32),
            pltpu.VMEM((K, 128), jnp.float32),
            pltpu.VMEM((ZB, 128), jnp.float32),
            pltpu.VMEM_SHARED((NA, 128), jnp.float32),
        ],
    )
    def k(dstd_hbm, out_hbm, dst_v, ones_v, zbuf, acc):
        cid = lax.axis_index("c")
        sid = lax.axis_index("s")
        _fill_rows(ones_v, K, 128, 1.0, jnp.float32)
        _fill_rows(zbuf, ZB, 128, 0.0, jnp.float32)
        _zero_acc_slice(zbuf, acc, sid)
        pltpu.sync_copy(dstd_hbm.at[cid * NS + sid], dst_v)
        plsc.subcore_barrier()

        def step(j, _):
            pltpu.sync_copy(ones_v, acc.at[dst_v.at[j]], add=True)
            return 0

        lax.fori_loop(0, CHD, step, 0)
        plsc.subcore_barrier()
        pltpu.sync_copy(acc.at[pl.ds(sid * RPT, RPT)],
                        out_hbm.at[pl.ds(cid * N + sid * RPT, RPT)])

        @pl.when(sid == NS - 1)
        def _():
            pltpu.sync_copy(acc.at[pl.ds(NS * RPT, TAIL)],
                            out_hbm.at[pl.ds(cid * N + NS * RPT, TAIL)])

    return k(dst_d)


def _prop_kernel(table, src_g, dst_t):
    @functools.partial(
        pl.kernel,
        out_type=jax.ShapeDtypeStruct((2 * N, 128), jnp.float32),
        mesh=_sc_mesh(),
        scratch_types=[
            pltpu.VMEM((CH, K), jnp.int32),
            pltpu.VMEM((CH, K), jnp.int32),
            pltpu.VMEM((K, 128), jnp.float32),
            pltpu.VMEM_SHARED((NA, 128), jnp.float32),
            pltpu.SemaphoreType.DMA,
        ],
    )
    def k(table_hbm, srcg_hbm, dstt_hbm, out_hbm,
          src_v, dst_v, rows_v, acc, gsem):
        cid = lax.axis_index("c")
        sid = lax.axis_index("s")
        _fill_rows(rows_v, K, 128, 0.0, jnp.float32)
        _zero_acc_slice(rows_v, acc, sid)
        pltpu.sync_copy(srcg_hbm.at[cid * NS + sid], src_v)
        pltpu.sync_copy(dstt_hbm.at[sid], dst_v)
        plsc.subcore_barrier()

        def step(j, _):
            pltpu.async_copy(table_hbm.at[src_v.at[j]], rows_v, gsem).wait()
            pltpu.sync_copy(rows_v, acc.at[dst_v.at[j]], add=True)
            return 0

        lax.fori_loop(0, CH, step, 0)
        plsc.subcore_barrier()
        pltpu.sync_copy(acc.at[pl.ds(sid * RPT, RPT)],
                        out_hbm.at[pl.ds(cid * N + sid * RPT, RPT)])

        @pl.when(sid == NS - 1)
        def _():
            pltpu.sync_copy(acc.at[pl.ds(NS * RPT, TAIL)],
                            out_hbm.at[pl.ds(cid * N + NS * RPT, TAIL)])

    return k(table, src_g, dst_t)


_R = 1000


def _dis_from_deg(deg_col):
    pos = deg_col > 0.0
    return jnp.where(pos, 1.0 / jnp.sqrt(jnp.where(pos, deg_col, 1.0)), 0.0)


def _tc_first(x, w0, deg2):
    def body(x_ref, w_ref, deg_ref, g_ref, dis_ref):
        deg = deg_ref[0, :, 0:1] + deg_ref[1, :, 0:1]
        dis = _dis_from_deg(deg)
        u = lax.dot_general(x_ref[...], w_ref[...], (((1,), (1,)), ((), ())),
                            preferred_element_type=jnp.float32)
        g = dis * u
        g_ref[0] = g[:, :128]
        g_ref[1] = g[:, 128:]
        dis_ref[...] = dis

    return pl.pallas_call(
        body,
        grid=(N // _R,),
        in_specs=[pl.BlockSpec((_R, 256), lambda i: (i, 0)),
                  pl.BlockSpec((256, 256), lambda i: (0, 0)),
                  pl.BlockSpec((2, _R, 128), lambda i: (0, i, 0))],
        out_specs=[pl.BlockSpec((2, _R, 128), lambda i: (0, i, 0)),
                   pl.BlockSpec((_R, 1), lambda i: (i, 0))],
        out_shape=[jax.ShapeDtypeStruct((2, N, 128), jnp.float32),
                   jax.ShapeDtypeStruct((N, 1), jnp.float32)],
    )(x, w0, deg2)


def _tc_mid_first(s, dis, b_prev, w):
    def body(s_ref, dis_ref, b_ref, w_ref, acc_ref, g_ref):
        dis = dis_ref[...]
        h = dis * jnp.concatenate([s_ref[0], s_ref[1]], axis=1) + b_ref[...]
        acc_ref[...] = h
        u = lax.dot_general(h, w_ref[...], (((1,), (1,)), ((), ())),
                            preferred_element_type=jnp.float32)
        g = dis * u
        g_ref[0] = g[:, :128]
        g_ref[1] = g[:, 128:]

    return pl.pallas_call(
        body,
        grid=(N // _R,),
        in_specs=[pl.BlockSpec((2, _R, 128), lambda i: (0, i, 0)),
                  pl.BlockSpec((_R, 1), lambda i: (i, 0)),
                  pl.BlockSpec((1, 256), lambda i: (0, 0)),
                  pl.BlockSpec((256, 256), lambda i: (0, 0))],
        out_specs=[pl.BlockSpec((_R, 256), lambda i: (i, 0)),
                   pl.BlockSpec((2, _R, 128), lambda i: (0, i, 0))],
        out_shape=[jax.ShapeDtypeStruct((N, 256), jnp.float32),
                   jax.ShapeDtypeStruct((2, N, 128), jnp.float32)],
    )(s, dis, b_prev, w)


def _tc_mid(s, dis, b_prev, w, acc_in):
    def body(s_ref, dis_ref, b_ref, w_ref, accin_ref, acc_ref, g_ref):
        dis = dis_ref[...]
        h = dis * jnp.concatenate([s_ref[0], s_ref[1]], axis=1) + b_ref[...]
        acc_ref[...] = accin_ref[...] + h
        u = lax.dot_general(h, w_ref[...], (((1,), (1,)), ((), ())),
                            preferred_element_type=jnp.float32)
        g = dis * u
        g_ref[0] = g[:, :128]
        g_ref[1] = g[:, 128:]

    return pl.pallas_call(
        body,
        grid=(N // _R,),
        in_specs=[pl.BlockSpec((2, _R, 128), lambda i: (0, i, 0)),
                  pl.BlockSpec((_R, 1), lambda i: (i, 0)),
                  pl.BlockSpec((1, 256), lambda i: (0, 0)),
                  pl.BlockSpec((256, 256), lambda i: (0, 0)),
                  pl.BlockSpec((_R, 256), lambda i: (i, 0))],
        out_specs=[pl.BlockSpec((_R, 256), lambda i: (i, 0)),
                   pl.BlockSpec((2, _R, 128), lambda i: (0, i, 0))],
        out_shape=[jax.ShapeDtypeStruct((N, 256), jnp.float32),
                   jax.ShapeDtypeStruct((2, N, 128), jnp.float32)],
    )(s, dis, b_prev, w, acc_in)


def _tc_last(s, dis, b_prev, acc_in, w_out, b_out):
    def body(s_ref, dis_ref, b_ref, accin_ref, w_ref, bout_ref, o_ref):
        h = dis_ref[...] * jnp.concatenate([s_ref[0], s_ref[1]], axis=1) + b_ref[...]
        m = (accin_ref[...] + h) * (1.0 / 3.0)
        o_ref[...] = lax.dot_general(
            m, w_ref[...], (((1,), (1,)), ((), ())),
            preferred_element_type=jnp.float32) + bout_ref[...]

    return pl.pallas_call(
        body,
        grid=(N // _R,),
        in_specs=[pl.BlockSpec((2, _R, 128), lambda i: (0, i, 0)),
                  pl.BlockSpec((_R, 1), lambda i: (i, 0)),
                  pl.BlockSpec((1, 256), lambda i: (0, 0)),
                  pl.BlockSpec((_R, 256), lambda i: (i, 0)),
                  pl.BlockSpec((128, 256), lambda i: (0, 0)),
                  pl.BlockSpec((1, 128), lambda i: (0, 0))],
        out_specs=pl.BlockSpec((_R, 128), lambda i: (i, 0)),
        out_shape=jax.ShapeDtypeStruct((N, 128), jnp.float32),
    )(s, dis, b_prev, acc_in, w_out, b_out)


def kernel(x, edge_index, W0, b0, W1, b1, W2, b2, W_out, b_out):
    src = edge_index[0]
    dst = edge_index[1]
    pad = CH * K - EPT
    junk = N + jnp.arange(pad, dtype=jnp.int32)
    src_p = jnp.concatenate(
        [src.reshape(NS, EPT), jnp.zeros((NS, pad), jnp.int32)], axis=1)
    dst_p = jnp.concatenate(
        [dst.reshape(NS, EPT), jnp.broadcast_to(junk, (NS, pad))], axis=1)
    src_g = jnp.concatenate([src_p, src_p + N]).reshape(2 * NS, CH, K)
    dst_t = dst_p.reshape(NS, CH, K)
    padd = CHD * K - EPD
    junkd = N + jnp.arange(padd, dtype=jnp.int32)
    dst_d = jnp.concatenate(
        [dst.reshape(NC * NS, EPD),
         jnp.broadcast_to(junkd, (NC * NS, padd))], axis=1)
    dst_d = dst_d.reshape(NC * NS, CHD, K)

    deg2 = _deg_kernel(dst_d).reshape(2, N, 128)
    g1, dis = _tc_first(x, W0, deg2)
    s1 = _prop_kernel(g1.reshape(2 * N, 128), src_g, dst_t).reshape(2, N, 128)
    acc1, g2 = _tc_mid_first(s1, dis, b0.reshape(1, -1), W1)
    s2 = _prop_kernel(g2.reshape(2 * N, 128), src_g, dst_t).reshape(2, N, 128)
    acc2, g3 = _tc_mid(s2, dis, b1.reshape(1, -1), W2, acc1)
    s3 = _prop_kernel(g3.reshape(2 * N, 128), src_g, dst_t).reshape(2, N, 128)
    return _tc_last(s3, dis, b2.reshape(1, -1), acc2, W_out, b_out.reshape(1, -1))


# speedup vs baseline: 1.6290x; 1.6290x over previous
import functools

import jax
import jax.numpy as jnp
from jax import lax
from jax.experimental import pallas as pl
from jax.experimental.pallas import tpu as pltpu
from jax.experimental.pallas import tpu_sc as plsc

N = 10000
E = 160000
NC = 2
NS = 16
EPT = E // NS
K = 125
CH = 80
EPD = E // (NC * NS)
CHD = 40
NA = N
RPT = 624
TAIL = N - NS * RPT
ZB = 104


def _fill_rows(ref, rows, cols, value, dtype):
    per_row = cols // 16

    def body(i, _):
        r = i // per_row
        c = (i % per_row) * 16
        ref[r, pl.ds(c, 16)] = jnp.full((16,), value, dtype)
        return 0

    lax.fori_loop(0, rows * per_row, body, 0)


def _sc_mesh():
    return plsc.VectorSubcoreMesh(core_axis_name="c", subcore_axis_name="s")


def _zero_acc_slice(zsrc, acc, sid):
    def zero_chunk(i, _):
        pltpu.sync_copy(zsrc.at[pl.ds(0, 96)],
                        acc.at[pl.ds(sid * RPT + i * 96, 96)])
        return 0

    lax.fori_loop(0, 6, zero_chunk, 0)
    pltpu.sync_copy(zsrc.at[pl.ds(0, 48)], acc.at[pl.ds(sid * RPT + 576, 48)])

    @pl.when(sid == NS - 1)
    def _():
        pltpu.sync_copy(zsrc.at[pl.ds(0, TAIL)], acc.at[pl.ds(NS * RPT, TAIL)])


def _deg_kernel(dst_d):
    @functools.partial(
        pl.kernel,
        out_type=jax.ShapeDtypeStruct((2 * N, 128), jnp.float32),
        mesh=_sc_mesh(),
        scratch_types=[
            pltpu.VMEM((CHD, K), jnp.int32),
            pltpu.VMEM((K, 128), jnp.float32),
            pltpu.VMEM((ZB, 128), jnp.float32),
            pltpu.VMEM_SHARED((NA, 128), jnp.float32),
        ],
    )
    def k(dstd_hbm, out_hbm, dst_v, ones_v, zbuf, acc):
        cid = lax.axis_index("c")
        sid = lax.axis_index("s")
        _fill_rows(ones_v, K, 128, 1.0, jnp.float32)
        _fill_rows(zbuf, ZB, 128, 0.0, jnp.float32)
        _zero_acc_slice(zbuf, acc, sid)
        pltpu.sync_copy(dstd_hbm.at[cid * NS + sid], dst_v)
        plsc.subcore_barrier()

        def step(j, _):
            pltpu.sync_copy(ones_v, acc.at[dst_v.at[j]], add=True)
            return 0

        lax.fori_loop(0, CHD, step, 0)
        plsc.subcore_barrier()
        pltpu.sync_copy(acc.at[pl.ds(sid * RPT, RPT)],
                        out_hbm.at[pl.ds(cid * N + sid * RPT, RPT)])

        @pl.when(sid == NS - 1)
        def _():
            pltpu.sync_copy(acc.at[pl.ds(NS * RPT, TAIL)],
                            out_hbm.at[pl.ds(cid * N + NS * RPT, TAIL)])

    return k(dst_d)


def _prop_kernel(table, src_g, dst_t):
    @functools.partial(
        pl.kernel,
        out_type=jax.ShapeDtypeStruct((2 * N, 128), jnp.float32),
        mesh=_sc_mesh(),
        scratch_types=[
            pltpu.VMEM((CH, K), jnp.int32),
            pltpu.VMEM((CH, K), jnp.int32),
            pltpu.VMEM((K, 128), jnp.float32),
            pltpu.VMEM_SHARED((NA, 128), jnp.float32),
            pltpu.SemaphoreType.DMA,
        ],
    )
    def k(table_hbm, srcg_hbm, dstt_hbm, out_hbm,
          src_v, dst_v, rows_v, acc, gsem):
        cid = lax.axis_index("c")
        sid = lax.axis_index("s")
        _fill_rows(rows_v, K, 128, 0.0, jnp.float32)
        _zero_acc_slice(rows_v, acc, sid)
        pltpu.sync_copy(srcg_hbm.at[cid * NS + sid], src_v)
        pltpu.sync_copy(dstt_hbm.at[sid], dst_v)
        plsc.subcore_barrier()

        def step(j, _):
            pltpu.async_copy(table_hbm.at[src_v.at[j]], rows_v, gsem).wait()
            pltpu.sync_copy(rows_v, acc.at[dst_v.at[j]], add=True)
            return 0

        lax.fori_loop(0, CH, step, 0)
        plsc.subcore_barrier()
        pltpu.sync_copy(acc.at[pl.ds(sid * RPT, RPT)],
                        out_hbm.at[pl.ds(cid * N + sid * RPT, RPT)])

        @pl.when(sid == NS - 1)
        def _():
            pltpu.sync_copy(acc.at[pl.ds(NS * RPT, TAIL)],
                            out_hbm.at[pl.ds(cid * N + NS * RPT, TAIL)])

    return k(table, src_g, dst_t)


_R = 1000


def _dis_from_deg(deg_col):
    pos = deg_col > 0.0
    return jnp.where(pos, 1.0 / jnp.sqrt(jnp.where(pos, deg_col, 1.0)), 0.0)


def _tc_first(x, w0, deg2):
    def body(x_ref, w_ref, deg_ref, g_ref, dis_ref):
        deg = deg_ref[0, :, 0:1] + deg_ref[1, :, 0:1]
        dis = _dis_from_deg(deg)
        u = lax.dot_general(x_ref[...], w_ref[...], (((1,), (1,)), ((), ())),
                            preferred_element_type=jnp.float32)
        g = dis * u
        g_ref[0] = g[:, :128]
        g_ref[1] = g[:, 128:]
        dis_ref[...] = dis

    return pl.pallas_call(
        body,
        grid=(N // _R,),
        in_specs=[pl.BlockSpec((_R, 256), lambda i: (i, 0)),
                  pl.BlockSpec((256, 256), lambda i: (0, 0)),
                  pl.BlockSpec((2, _R, 128), lambda i: (0, i, 0))],
        out_specs=[pl.BlockSpec((2, _R, 128), lambda i: (0, i, 0)),
                   pl.BlockSpec((_R, 1), lambda i: (i, 0))],
        out_shape=[jax.ShapeDtypeStruct((2, N, 128), jnp.float32),
                   jax.ShapeDtypeStruct((N, 1), jnp.float32)],
    )(x, w0, deg2)


def _tc_mid_first(s, dis, b_prev, w):
    def body(s_ref, dis_ref, b_ref, w_ref, acc_ref, g_ref):
        dis = dis_ref[...]
        h = dis * jnp.concatenate([s_ref[0], s_ref[1]], axis=1) + b_ref[...]
        acc_ref[...] = h
        u = lax.dot_general(h, w_ref[...], (((1,), (1,)), ((), ())),
                            preferred_element_type=jnp.float32)
        g = dis * u
        g_ref[0] = g[:, :128]
        g_ref[1] = g[:, 128:]

    return pl.pallas_call(
        body,
        grid=(N // _R,),
        in_specs=[pl.BlockSpec((2, _R, 128), lambda i: (0, i, 0)),
                  pl.BlockSpec((_R, 1), lambda i: (i, 0)),
                  pl.BlockSpec((1, 256), lambda i: (0, 0)),
                  pl.BlockSpec((256, 256), lambda i: (0, 0))],
        out_specs=[pl.BlockSpec((_R, 256), lambda i: (i, 0)),
                   pl.BlockSpec((2, _R, 128), lambda i: (0, i, 0))],
        out_shape=[jax.ShapeDtypeStruct((N, 256), jnp.float32),
                   jax.ShapeDtypeStruct((2, N, 128), jnp.float32)],
    )(s, dis, b_prev, w)


def _tc_mid(s, dis, b_prev, w, acc_in):
    def body(s_ref, dis_ref, b_ref, w_ref, accin_ref, acc_ref, g_ref):
        dis = dis_ref[...]
        h = dis * jnp.concatenate([s_ref[0], s_ref[1]], axis=1) + b_ref[...]
        acc_ref[...] = accin_ref[...] + h
        u = lax.dot_general(h, w_ref[...], (((1,), (1,)), ((), ())),
                            preferred_element_type=jnp.float32)
        g = dis * u
        g_ref[0] = g[:, :128]
        g_ref[1] = g[:, 128:]

    return pl.pallas_call(
        body,
        grid=(N // _R,),
        in_specs=[pl.BlockSpec((2, _R, 128), lambda i: (0, i, 0)),
                  pl.BlockSpec((_R, 1), lambda i: (i, 0)),
                  pl.BlockSpec((1, 256), lambda i: (0, 0)),
                  pl.BlockSpec((256, 256), lambda i: (0, 0)),
                  pl.BlockSpec((_R, 256), lambda i: (i, 0))],
        out_specs=[pl.BlockSpec((_R, 256), lambda i: (i, 0)),
                   pl.BlockSpec((2, _R, 128), lambda i: (0, i, 0))],
        out_shape=[jax.ShapeDtypeStruct((N, 256), jnp.float32),
                   jax.ShapeDtypeStruct((2, N, 128), jnp.float32)],
    )(s, dis, b_prev, w, acc_in)


def _tc_last(s, dis, b_prev, acc_in, w_out, b_out):
    def body(s_ref, dis_ref, b_ref, accin_ref, w_ref, bout_ref, o_ref):
        h = dis_ref[...] * jnp.concatenate([s_ref[0], s_ref[1]], axis=1) + b_ref[...]
        m = (accin_ref[...] + h) * (1.0 / 3.0)
        o_ref[...] = lax.dot_general(
            m, w_ref[...], (((1,), (1,)), ((), ())),
            preferred_element_type=jnp.float32) + bout_ref[...]

    return pl.pallas_call(
        body,
        grid=(N // _R,),
        in_specs=[pl.BlockSpec((2, _R, 128), lambda i: (0, i, 0)),
                  pl.BlockSpec((_R, 1), lambda i: (i, 0)),
                  pl.BlockSpec((1, 256), lambda i: (0, 0)),
                  pl.BlockSpec((_R, 256), lambda i: (i, 0)),
                  pl.BlockSpec((128, 256), lambda i: (0, 0)),
                  pl.BlockSpec((1, 128), lambda i: (0, 0))],
        out_specs=pl.BlockSpec((_R, 128), lambda i: (i, 0)),
        out_shape=jax.ShapeDtypeStruct((N, 128), jnp.float32),
    )(s, dis, b_prev, acc_in, w_out, b_out)


def kernel(x, edge_index, W0, b0, W1, b1, W2, b2, W_out, b_out):
    src = edge_index[0]
    dst = edge_index[1]
    src_p = src.reshape(NS, EPT)
    src_g = jnp.concatenate([src_p, src_p + N]).reshape(2 * NS, CH, K)
    dst_t = dst.reshape(NS, CH, K)
    dst_d = dst.reshape(NC * NS, CHD, K)

    deg2 = _deg_kernel(dst_d).reshape(2, N, 128)
    g1, dis = _tc_first(x, W0, deg2)
    s1 = _prop_kernel(g1.reshape(2 * N, 128), src_g, dst_t).reshape(2, N, 128)
    acc1, g2 = _tc_mid_first(s1, dis, b0.reshape(1, -1), W1)
    s2 = _prop_kernel(g2.reshape(2 * N, 128), src_g, dst_t).reshape(2, N, 128)
    acc2, g3 = _tc_mid(s2, dis, b1.reshape(1, -1), W2, acc1)
    s3 = _prop_kernel(g3.reshape(2 * N, 128), src_g, dst_t).reshape(2, N, 128)
    return _tc_last(s3, dis, b2.reshape(1, -1), acc2, W_out, b_out.reshape(1, -1))
